# Initial kernel scaffold; baseline (speedup 1.0000x reference)
import jax
import jax.numpy as jnp
from jax.experimental import pallas as pl


def kernel(h, W_e, b_e, W_g):
    raise NotImplementedError("write your pallas kernel here")


# speedup vs baseline: 1.7510x; 1.7510x over previous
import jax
import jax.numpy as jnp
from jax.experimental import pallas as pl
from jax.experimental.pallas import tpu as pltpu

_HID = 4096
_K = 64
_TOP_K = 8
_TB = 256


def _body(h_ref, wt_ref, b_ref, o_ref):
    z = jnp.dot(h_ref[...], wt_ref[...], preferred_element_type=jnp.float32)
    z = z + b_ref[...]
    tb = z.shape[0]
    lane = jax.lax.broadcasted_iota(jnp.int32, (tb, 2 * _K), 1)
    neg = jnp.float32(-jnp.inf)
    work = jnp.where(lane < _K, z, neg)

    m = jnp.max(work, axis=1, keepdims=True)
    cur = m
    for _ in range(_TOP_K - 1):
        work = jnp.where(work == cur, neg, work)
        cur = jnp.max(work, axis=1, keepdims=True)
    thresh = cur

    g = jnp.where(lane < _K, z, neg)
    p = jnp.where(g >= thresh, jnp.exp(g - m), 0.0)
    den = jnp.sum(p, axis=1, keepdims=True)
    p_shift = jnp.roll(p, _K, axis=1)
    num = jnp.sum(p_shift * z, axis=1, keepdims=True)
    res = num / den
    o_ref[...] = jax.lax.transpose(res, (1, 0)).reshape(1, 1, tb)


def kernel(h, W_e, b_e, W_g):
    B, L, _ = h.shape
    n_tok = B * L
    hf = h.reshape(n_tok, _HID)
    wt = jnp.concatenate([W_g, W_e], axis=0).T
    bias = jnp.concatenate([jnp.zeros((_K,), b_e.dtype), b_e]).reshape(1, 2 * _K)
    nb = n_tok // _TB

    out = pl.pallas_call(
        _body,
        grid=(nb,),
        in_specs=[
            pl.BlockSpec((_TB, _HID), lambda i: (i, 0)),
            pl.BlockSpec((_HID, 2 * _K), lambda i: (0, 0)),
            pl.BlockSpec((1, 2 * _K), lambda i: (0, 0)),
        ],
        out_specs=pl.BlockSpec((1, 1, _TB), lambda i: (i, 0, 0)),
        out_shape=jax.ShapeDtypeStruct((nb, 1, _TB), jnp.float32),
    )(hf, wt, bias)
    return out.reshape(B, L)


# speedup vs baseline: 2.1688x; 1.2386x over previous
import jax
import jax.numpy as jnp
from jax.experimental import pallas as pl
from jax.experimental.pallas import tpu as pltpu

_HID = 4096
_K = 64
_TOP_K = 8
_TB = 512


def _body(h_ref, wt_ref, b_ref, o_ref):
    z = jnp.dot(h_ref[...], wt_ref[...], preferred_element_type=jnp.float32)
    z = z + b_ref[...]
    tb = z.shape[0]
    lane = jax.lax.broadcasted_iota(jnp.int32, (tb, 2 * _K), 1)
    neg = jnp.float32(-jnp.inf)
    work = jnp.where(lane < _K, z, neg)

    m = jnp.max(work, axis=1, keepdims=True)
    cur = m
    for _ in range(_TOP_K - 1):
        work = jnp.where(work == cur, neg, work)
        cur = jnp.max(work, axis=1, keepdims=True)
    thresh = cur

    g = jnp.where(lane < _K, z, neg)
    p = jnp.where(g >= thresh, jnp.exp(g - m), 0.0)
    den = jnp.sum(p, axis=1, keepdims=True)
    p_shift = jnp.roll(p, _K, axis=1)
    num = jnp.sum(p_shift * z, axis=1, keepdims=True)
    res = num / den
    o_ref[...] = jax.lax.transpose(res, (1, 0)).reshape(1, 1, tb)


def kernel(h, W_e, b_e, W_g):
    B, L, _ = h.shape
    n_tok = B * L
    hf = h.reshape(n_tok, _HID)
    wt = jnp.concatenate([W_g, W_e], axis=0).T
    bias = jnp.concatenate([jnp.zeros((_K,), b_e.dtype), b_e]).reshape(1, 2 * _K)
    nb = n_tok // _TB

    out = pl.pallas_call(
        _body,
        grid=(nb,),
        in_specs=[
            pl.BlockSpec((_TB, _HID), lambda i: (i, 0)),
            pl.BlockSpec((_HID, 2 * _K), lambda i: (0, 0)),
            pl.BlockSpec((1, 2 * _K), lambda i: (0, 0)),
        ],
        out_specs=pl.BlockSpec((1, 1, _TB), lambda i: (i, 0, 0)),
        out_shape=jax.ShapeDtypeStruct((nb, 1, _TB), jnp.float32),
    )(hf, wt, bias)
    return out.reshape(B, L)


# speedup vs baseline: 2.4024x; 1.1077x over previous
import jax
import jax.numpy as jnp
from jax.experimental import pallas as pl
from jax.experimental.pallas import tpu as pltpu

_HID = 4096
_K = 64
_TOP_K = 8
_TB = 1024


def _body(h_ref, wt_ref, b_ref, o_ref):
    z = jnp.dot(h_ref[...], wt_ref[...], preferred_element_type=jnp.float32)
    z = z + b_ref[...]
    tb = z.shape[0]
    lane = jax.lax.broadcasted_iota(jnp.int32, (tb, 2 * _K), 1)
    neg = jnp.float32(-jnp.inf)
    work = jnp.where(lane < _K, z, neg)

    m = jnp.max(work, axis=1, keepdims=True)
    cur = m
    for _ in range(_TOP_K - 1):
        work = jnp.where(work == cur, neg, work)
        cur = jnp.max(work, axis=1, keepdims=True)
    thresh = cur

    g = jnp.where(lane < _K, z, neg)
    p = jnp.where(g >= thresh, jnp.exp(g - m), 0.0)
    den = jnp.sum(p, axis=1, keepdims=True)
    p_shift = jnp.roll(p, _K, axis=1)
    num = jnp.sum(p_shift * z, axis=1, keepdims=True)
    res = num / den
    o_ref[...] = jax.lax.transpose(res, (1, 0)).reshape(1, 1, tb)


def kernel(h, W_e, b_e, W_g):
    B, L, _ = h.shape
    n_tok = B * L
    hf = h.reshape(n_tok, _HID)
    wt = jnp.concatenate([W_g, W_e], axis=0).T
    bias = jnp.concatenate([jnp.zeros((_K,), b_e.dtype), b_e]).reshape(1, 2 * _K)
    nb = n_tok // _TB

    out = pl.pallas_call(
        _body,
        grid=(nb,),
        in_specs=[
            pl.BlockSpec((_TB, _HID), lambda i: (i, 0)),
            pl.BlockSpec((_HID, 2 * _K), lambda i: (0, 0)),
            pl.BlockSpec((1, 2 * _K), lambda i: (0, 0)),
        ],
        out_specs=pl.BlockSpec((1, 1, _TB), lambda i: (i, 0, 0)),
        out_shape=jax.ShapeDtypeStruct((nb, 1, _TB), jnp.float32),
    )(hf, wt, bias)
    return out.reshape(B, L)
